# Initial kernel scaffold; baseline (speedup 1.0000x reference)
import jax
import jax.numpy as jnp
from jax.experimental import pallas as pl


def kernel(indices, weight):
    raise NotImplementedError("write your pallas kernel here")


# speedup vs baseline: 1.9232x; 1.9232x over previous
import jax
import jax.numpy as jnp
from jax import lax
from jax.experimental import pallas as pl
from jax.experimental.pallas import tpu as pltpu
from jax.experimental.pallas import tpu_sc as plsc

_MAX_NORM = 0.95
_L = 16
_D = 128
_NV = _D // _L

_NC = 2
_NS = 16
_NW = _NC * _NS

_B = 4096 * 50
_BPW = _B // _NW
_CHUNK = 128
_NCH = _BPW // _CHUNK


def _rsqrt_nr(x):
    i = plsc.bitcast(x, jnp.int32)
    y = plsc.bitcast(jnp.int32(0x5F3759DF) - (i >> 1), jnp.float32)
    for _ in range(2):
        y = y * (1.5 - 0.5 * x * y * y)
    return y


def _project_rows(buf):
    def row_fn(r, carry):
        vs = [buf[r, pl.ds(k * _L, _L)] for k in range(_NV)]
        acc = vs[0] * vs[0]
        for k in range(1, _NV):
            acc = acc + vs[k] * vs[k]
        ss = jnp.full((_L,), jnp.sum(acc), jnp.float32)
        scale = jnp.minimum(_MAX_NORM * _rsqrt_nr(ss), 1.0)
        for k in range(_NV):
            buf[r, pl.ds(k * _L, _L)] = vs[k] * scale
        return carry

    lax.fori_loop(0, _CHUNK, row_fn, 0)


def _sc_body(idx_hbm, table_hbm, out_hbm, idx_v, buf, gsem):
    wid = lax.axis_index("s") * _NC + lax.axis_index("c")
    base = wid * _BPW
    pltpu.sync_copy(idx_hbm.at[wid], idx_v)

    def chunk_fn(j, carry):
        pltpu.async_copy(table_hbm.at[idx_v.at[j]], buf, gsem).wait()
        _project_rows(buf)
        pltpu.sync_copy(buf, out_hbm.at[pl.ds(base + j * _CHUNK, _CHUNK)])
        return carry

    lax.fori_loop(0, _NCH, chunk_fn, 0)


def kernel(indices, weight):
    idx = indices.astype(jnp.int32).reshape(_NW, _NCH, _CHUNK)
    kfn = pl.kernel(
        _sc_body,
        mesh=plsc.VectorSubcoreMesh(core_axis_name="c", subcore_axis_name="s"),
        out_type=jax.ShapeDtypeStruct((_B, _D), jnp.float32),
        scratch_types=[
            pltpu.VMEM((_NCH, _CHUNK), jnp.int32),
            pltpu.VMEM((_CHUNK, _D), jnp.float32),
            pltpu.SemaphoreType.DMA,
        ],
        compiler_params=pltpu.CompilerParams(needs_layout_passes=False),
    )
    out = kfn(idx, weight)
    return out.reshape(indices.shape + (_D,))


# speedup vs baseline: 3.5476x; 1.8447x over previous
import jax
import jax.numpy as jnp
from jax import lax
from jax.experimental import pallas as pl
from jax.experimental.pallas import tpu as pltpu
from jax.experimental.pallas import tpu_sc as plsc

_MAX_NORM = 0.95
_L = 16
_D = 128
_NV = _D // _L

_NC = 2
_NS = 16
_NW = _NC * _NS

_B = 4096 * 50
_BPW = _B // _NW
_CHUNK = 128
_NCH = _BPW // _CHUNK
_NBUF = 4


def _rsqrt_nr(x):
    i = plsc.bitcast(x, jnp.int32)
    y = plsc.bitcast(jnp.int32(0x5F3759DF) - (i >> 1), jnp.float32)
    xh = 0.5 * x
    for _ in range(2):
        y = y * (1.5 - xh * y * y)
    return y


def _project_rows(buf):
    def row_fn(r, carry):
        vs = [buf[r, pl.ds(k * _L, _L)] for k in range(_NV)]
        acc = vs[0] * vs[0]
        for k in range(1, _NV):
            acc = acc + vs[k] * vs[k]
        ss = jnp.full((_L,), jnp.sum(acc), jnp.float32)
        scale = jnp.minimum(_MAX_NORM * _rsqrt_nr(ss), 1.0)
        for k in range(_NV):
            buf[r, pl.ds(k * _L, _L)] = vs[k] * scale
        return carry

    lax.fori_loop(0, _CHUNK, row_fn, 0, unroll=4)


def _sc_body(idx_hbm, table_hbm, out_hbm, idx_v, bufs, gsems, osems):
    wid = lax.axis_index("s") * _NC + lax.axis_index("c")
    base = wid * _BPW
    pltpu.sync_copy(idx_hbm.at[wid], idx_v)

    def gather(j, s):
        return pltpu.make_async_copy(table_hbm.at[idx_v.at[j]], bufs[s], gsems[s])

    def out_cp(j, s):
        return pltpu.make_async_copy(
            bufs[s], out_hbm.at[pl.ds(base + j * _CHUNK, _CHUNK)], osems[s])

    for s in range(_NBUF - 1):
        gather(s, s).start()

    def body(t, carry):
        for b in range(_NBUF):
            j = _NBUF * t + b
            gather(j, b).wait()
            _project_rows(bufs[b])
            out_cp(j, b).start()
            s_next = (b + _NBUF - 1) % _NBUF
            jj = j - 1

            @pl.when(jj >= 0)
            def _wait_prev():
                out_cp(jj, s_next).wait()

            g = j + _NBUF - 1

            @pl.when(g < _NCH)
            def _refill():
                gather(g, s_next).start()
        return carry

    n_full = _NCH // _NBUF
    lax.fori_loop(0, n_full, body, 0)

    for b in range(_NCH - n_full * _NBUF):
        j = n_full * _NBUF + b
        gather(j, b).wait()
        _project_rows(bufs[b])
        out_cp(j, b).start()
        out_cp(j - 1, (b + _NBUF - 1) % _NBUF).wait()
    out_cp(_NCH - 1, (_NCH - 1) % _NBUF).wait()


def kernel(indices, weight):
    idx = indices.astype(jnp.int32).reshape(_NW, _NCH, _CHUNK)
    kfn = pl.kernel(
        lambda ih, th, oh, iv, b0, b1, b2, b3, g0, g1, g2, g3, o0, o1, o2, o3:
            _sc_body(ih, th, oh, iv, (b0, b1, b2, b3),
                     (g0, g1, g2, g3), (o0, o1, o2, o3)),
        mesh=plsc.VectorSubcoreMesh(core_axis_name="c", subcore_axis_name="s"),
        out_type=jax.ShapeDtypeStruct((_B, _D), jnp.float32),
        scratch_types=(
            [pltpu.VMEM((_NCH, _CHUNK), jnp.int32)]
            + [pltpu.VMEM((_CHUNK, _D), jnp.float32)] * _NBUF
            + [pltpu.SemaphoreType.DMA] * (2 * _NBUF)
        ),
        compiler_params=pltpu.CompilerParams(needs_layout_passes=False),
    )
    out = kfn(idx, weight)
    return out.reshape(indices.shape + (_D,))


# speedup vs baseline: 10.7662x; 3.0348x over previous
import jax
import jax.numpy as jnp
from jax import lax
from jax.experimental import pallas as pl
from jax.experimental.pallas import tpu as pltpu
from jax.experimental.pallas import tpu_sc as plsc

_MAX_NORM = 0.95
_L = 16
_D = 128
_NV = _D // _L

_NC = 2
_NS = 16
_NW = _NC * _NS

_B = 4096 * 50
_BPW = _B // _NW
_CHUNK = 128
_NCH = _BPW // _CHUNK
_NBUF = 4


def _rsqrt_nr(x):
    i = plsc.bitcast(x, jnp.int32)
    y = plsc.bitcast(jnp.int32(0x5F3759DF) - (i >> 1), jnp.float32)
    xh = 0.5 * x
    for _ in range(2):
        y = y * (1.5 - xh * y * y)
    return y


def _project_rows(buf):
    def row_fn(r, carry):
        vs = [buf[r, pl.ds(k * _L, _L)] for k in range(_NV)]
        acc = vs[0] * vs[0]
        for k in range(1, _NV):
            acc = acc + vs[k] * vs[k]
        ss = jnp.full((_L,), jnp.sum(acc), jnp.float32)
        scale = jnp.minimum(_MAX_NORM * _rsqrt_nr(ss), 1.0)
        for k in range(_NV):
            buf[r, pl.ds(k * _L, _L)] = vs[k] * scale
        return carry

    lax.fori_loop(0, _CHUNK, row_fn, 0, unroll=4)


def _sc_body(idx_hbm, table_hbm, out_hbm, idx_v, bufs, gsems, osems):
    wid = lax.axis_index("s") * _NC + lax.axis_index("c")
    base = wid * _BPW
    pltpu.sync_copy(idx_hbm.at[wid], idx_v)

    def gather(j, s):
        return pltpu.make_async_copy(table_hbm.at[idx_v.at[j]], bufs[s], gsems[s])

    def out_cp(j, s):
        return pltpu.make_async_copy(
            bufs[s], out_hbm.at[pl.ds(base + j * _CHUNK, _CHUNK)], osems[s])

    for s in range(_NBUF - 1):
        gather(s, s).start()

    def body(t, carry):
        for b in range(_NBUF):
            j = _NBUF * t + b
            gather(j, b).wait()
            _project_rows(bufs[b])
            out_cp(j, b).start()
            s_next = (b + _NBUF - 1) % _NBUF
            jj = j - 1

            @pl.when(jj >= 0)
            def _wait_prev():
                out_cp(jj, s_next).wait()

            g = j + _NBUF - 1

            @pl.when(g < _NCH)
            def _refill():
                gather(g, s_next).start()
        return carry

    n_full = _NCH // _NBUF
    lax.fori_loop(0, n_full, body, 0)

    for b in range(_NCH - n_full * _NBUF):
        j = n_full * _NBUF + b
        gather(j, b).wait()
        _project_rows(bufs[b])
        out_cp(j, b).start()
        out_cp(j - 1, (b + _NBUF - 1) % _NBUF).wait()
    out_cp(_NCH - 1, (_NCH - 1) % _NBUF).wait()


def kernel(indices, weight):
    nb, ns = indices.shape
    idx = indices.astype(jnp.int32).T.reshape(_NW, _NCH, _CHUNK)
    kfn = pl.kernel(
        lambda ih, th, oh, iv, b0, b1, b2, b3, g0, g1, g2, g3, o0, o1, o2, o3:
            _sc_body(ih, th, oh, iv, (b0, b1, b2, b3),
                     (g0, g1, g2, g3), (o0, o1, o2, o3)),
        mesh=plsc.VectorSubcoreMesh(core_axis_name="c", subcore_axis_name="s"),
        out_type=jax.ShapeDtypeStruct((_B, _D), jnp.float32),
        scratch_types=(
            [pltpu.VMEM((_NCH, _CHUNK), jnp.int32)]
            + [pltpu.VMEM((_CHUNK, _D), jnp.float32)] * _NBUF
            + [pltpu.SemaphoreType.DMA] * (2 * _NBUF)
        ),
        compiler_params=pltpu.CompilerParams(needs_layout_passes=False),
    )
    out = kfn(idx, weight)
    return out.reshape(ns, nb, _D).transpose(1, 0, 2)
